# baseline (speedup 1.0000x reference)
import functools

import jax
import jax.numpy as jnp
from jax import lax
from jax.experimental import pallas as pl
from jax.experimental.pallas import tpu as pltpu
from jax.experimental.pallas import tpu_sc as plsc

_BATCH = 16384
_FEATURES = 512
_PATHWAYS = 64
_GROUP = 8

_NUM_CORES = 2
_NUM_SUBCORES = 16
_NUM_WORKERS = _NUM_CORES * _NUM_SUBCORES
_ROWS_PER_WORKER = _BATCH // _NUM_WORKERS
_CHUNK = 64
_NUM_CHUNKS = _ROWS_PER_WORKER // _CHUNK
_LANES = 16


def _body(x_hbm, out_hbm, in_v0, in_v1, out_v, si0, si1, so):
    wid = lax.axis_index("s") * _NUM_CORES + lax.axis_index("c")
    row0 = wid * _ROWS_PER_WORKER

    in_bufs = [in_v0, in_v1]
    in_sems = [si0, si1]

    lane = lax.iota(jnp.int32, _LANES)
    cv = (lane // _GROUP) * 4096 + (lane % _GROUP) * 128

    def start_in(c):
        rowa = row0 + c * _CHUNK
        return pltpu.async_copy(
            x_hbm.at[pl.ds(rowa * _FEATURES, _CHUNK * _FEATURES)],
            in_bufs[c & 1],
            in_sems[c & 1],
        )

    in_descs = [start_in(0), None]
    for c in range(_NUM_CHUNKS):
        b = c & 1
        if c + 1 < _NUM_CHUNKS:
            in_descs[1 - b] = start_in(c + 1)
        in_descs[b].wait()
        in_v = in_bufs[b]

        @plsc.parallel_loop(0, _PATHWAYS, 1, unroll=2)
        def do_pathway(p):
            bvec = cv + jnp.full(
                (_LANES,), (p // 16) * 1024 + (p % 16) * _GROUP, dtype=jnp.int32
            )
            pso = (p // _GROUP) * 4096 + (p % _GROUP) * 128
            for rl0 in (0, 16, 32, 48):
                base = (rl0 // _GROUP) * 4096
                acc = plsc.load_gather(in_v, [bvec + base])
                for k in range(1, _GROUP):
                    acc = acc + plsc.load_gather(in_v, [bvec + (base + k)])
                g = c * _CHUNK + rl0
                so_off = (g // 128) * 1024 + (g % 128)
                out_v[pl.ds(pso + so_off, _LANES)] = acc * (1.0 / _GROUP)

    out_descs = []
    for p8 in range(_PATHWAYS // _GROUP):
        src = out_v.at[pl.ds(p8 * 4096, 4096)]
        dst = out_hbm.at[pl.ds((p8 * 128 + wid * 4) * 1024, 4096)]
        out_descs.append(pltpu.async_copy(src, dst, so))
    for d in out_descs:
        d.wait()


@jax.jit
def kernel(gene_set_features):
    mesh = plsc.VectorSubcoreMesh(core_axis_name="c", subcore_axis_name="s")
    run = functools.partial(
        pl.kernel,
        out_type=jax.ShapeDtypeStruct((_BATCH * _PATHWAYS,), jnp.float32),
        mesh=mesh,
        scratch_types=[
            pltpu.VMEM((_CHUNK * _FEATURES,), jnp.float32),
            pltpu.VMEM((_CHUNK * _FEATURES,), jnp.float32),
            pltpu.VMEM((_ROWS_PER_WORKER * _PATHWAYS,), jnp.float32),
            pltpu.SemaphoreType.DMA,
            pltpu.SemaphoreType.DMA,
            pltpu.SemaphoreType.DMA,
        ],
        compiler_params=pltpu.CompilerParams(needs_layout_passes=False),
    )(_body)

    x4 = gene_set_features.reshape(_BATCH // 8, 8, _FEATURES // 128, 128)
    x_flat = x4.transpose(0, 2, 1, 3).reshape(-1)

    flat = run(x_flat)

    o4 = flat.reshape(_PATHWAYS // 8, _BATCH // 128, 8, 128)
    return o4.transpose(1, 3, 0, 2).reshape(_BATCH, _PATHWAYS)


# speedup vs baseline: 2.0403x; 2.0403x over previous
import functools

import jax
import jax.numpy as jnp
from jax import lax
from jax.experimental import pallas as pl
from jax.experimental.pallas import tpu as pltpu
from jax.experimental.pallas import tpu_sc as plsc

_BATCH = 16384
_FEATURES = 512
_PATHWAYS = 64
_GROUP = 8

_NUM_CORES = 2
_NUM_SUBCORES = 16
_NUM_WORKERS = _NUM_CORES * _NUM_SUBCORES
_ROWS_PER_WORKER = _BATCH // _NUM_WORKERS
_CHUNK = 64
_NUM_CHUNKS = _ROWS_PER_WORKER // _CHUNK
_LANES = 16
_QVECS = _PATHWAYS // _LANES


def _body(x_hbm, out_hbm, in_v0, in_v1, out_v0, out_v1, si0, si1, so0, so1):
    wid = lax.axis_index("s") * _NUM_CORES + lax.axis_index("c")
    row0 = wid * _ROWS_PER_WORKER

    in_bufs = [in_v0, in_v1]
    out_bufs = [out_v0, out_v1]
    in_sems = [si0, si1]
    out_sems = [so0, so1]

    lane = lax.iota(jnp.int32, _LANES)
    col_idx = [
        [lane * _GROUP + (q * 1024 + k) for k in range(_GROUP)]
        for q in range(_QVECS)
    ]

    def start_in(c):
        rowa = row0 + c * _CHUNK
        return pltpu.async_copy(
            x_hbm.at[pl.ds(rowa * _FEATURES, _CHUNK * _FEATURES)],
            in_bufs[c & 1],
            in_sems[c & 1],
        )

    in_descs = [start_in(0), None]
    out_descs = [None, None]
    for c in range(_NUM_CHUNKS):
        b = c & 1
        if c + 1 < _NUM_CHUNKS:
            in_descs[1 - b] = start_in(c + 1)
        in_descs[b].wait()
        if out_descs[b] is not None:
            out_descs[b].wait()
        in_v = in_bufs[b]
        out_v = out_bufs[b]

        @plsc.parallel_loop(0, _CHUNK, 1, unroll=2)
        def do_row(r):
            rbase = jnp.full(
                (_LANES,), (r // 8) * 4096 + (r % 8) * 128, dtype=jnp.int32
            )
            for q in range(_QVECS):
                acc = plsc.load_gather(in_v, [rbase + col_idx[q][0]])
                for k in range(1, _GROUP):
                    acc = acc + plsc.load_gather(in_v, [rbase + col_idx[q][k]])
                out_v[pl.ds(r * _PATHWAYS + q * _LANES, _LANES)] = acc * (
                    1.0 / _GROUP
                )

        rowa = row0 + c * _CHUNK
        out_descs[b] = pltpu.async_copy(
            out_v,
            out_hbm.at[pl.ds(rowa * _PATHWAYS, _CHUNK * _PATHWAYS)],
            out_sems[b],
        )
    out_descs[0].wait()
    out_descs[1].wait()


@jax.jit
def kernel(gene_set_features):
    mesh = plsc.VectorSubcoreMesh(core_axis_name="c", subcore_axis_name="s")
    run = functools.partial(
        pl.kernel,
        out_type=jax.ShapeDtypeStruct((_BATCH * _PATHWAYS,), jnp.float32),
        mesh=mesh,
        scratch_types=[
            pltpu.VMEM((_CHUNK * _FEATURES,), jnp.float32),
            pltpu.VMEM((_CHUNK * _FEATURES,), jnp.float32),
            pltpu.VMEM((_CHUNK * _PATHWAYS,), jnp.float32),
            pltpu.VMEM((_CHUNK * _PATHWAYS,), jnp.float32),
            pltpu.SemaphoreType.DMA,
            pltpu.SemaphoreType.DMA,
            pltpu.SemaphoreType.DMA,
            pltpu.SemaphoreType.DMA,
        ],
        compiler_params=pltpu.CompilerParams(needs_layout_passes=False),
    )(_body)

    x4 = gene_set_features.reshape(_BATCH // 8, 8, _FEATURES // 128, 128)
    x_flat = x4.transpose(0, 2, 1, 3).reshape(-1)

    flat = run(x_flat)
    return flat.reshape(_BATCH, _PATHWAYS)


# speedup vs baseline: 2.8521x; 1.3979x over previous
import functools

import jax
import jax.numpy as jnp
from jax import lax
from jax.experimental import pallas as pl
from jax.experimental.pallas import tpu as pltpu
from jax.experimental.pallas import tpu_sc as plsc

_BATCH = 16384
_FEATURES = 512
_PATHWAYS = 64
_GROUP = 8

_SC_ROWS = 4096
_NUM_CORES = 2
_NUM_SUBCORES = 16
_NUM_WORKERS = _NUM_CORES * _NUM_SUBCORES
_ROWS_PER_WORKER = _SC_ROWS // _NUM_WORKERS
_CHUNK = 64
_NUM_CHUNKS = _ROWS_PER_WORKER // _CHUNK
_TC_BLOCK = 4096
_LANES = 16
_QVECS = _PATHWAYS // _LANES


def _sc_body(x_hbm, out_hbm, in_v0, in_v1, out_v0, out_v1, si0, si1, so0, so1):
    wid = lax.axis_index("s") * _NUM_CORES + lax.axis_index("c")
    row0 = wid * _ROWS_PER_WORKER

    in_bufs = [in_v0, in_v1]
    out_bufs = [out_v0, out_v1]
    in_sems = [si0, si1]
    out_sems = [so0, so1]

    lane = lax.iota(jnp.int32, _LANES)
    col_idx = [
        [lane * _GROUP + (q * 1024 + k) for k in range(_GROUP)]
        for q in range(_QVECS)
    ]

    def start_in(c):
        rowa = row0 + c * _CHUNK
        return pltpu.async_copy(
            x_hbm.at[pl.ds(rowa * _FEATURES, _CHUNK * _FEATURES)],
            in_bufs[c & 1],
            in_sems[c & 1],
        )

    in_descs = [start_in(0), None]
    out_descs = [None, None]
    for c in range(_NUM_CHUNKS):
        b = c & 1
        if c + 1 < _NUM_CHUNKS:
            in_descs[1 - b] = start_in(c + 1)
        in_descs[b].wait()
        if out_descs[b] is not None:
            out_descs[b].wait()
        in_v = in_bufs[b]
        out_v = out_bufs[b]

        @plsc.parallel_loop(0, _CHUNK, 1, unroll=2)
        def do_row(r):
            rbase = jnp.full(
                (_LANES,), (r // 8) * 4096 + (r % 8) * 128, dtype=jnp.int32
            )
            for q in range(_QVECS):
                acc = plsc.load_gather(in_v, [rbase + col_idx[q][0]])
                for k in range(1, _GROUP):
                    acc = acc + plsc.load_gather(in_v, [rbase + col_idx[q][k]])
                out_v[pl.ds(r * _PATHWAYS + q * _LANES, _LANES)] = acc * (
                    1.0 / _GROUP
                )

        rowa = row0 + c * _CHUNK
        out_descs[b] = pltpu.async_copy(
            out_v,
            out_hbm.at[pl.ds(rowa * _PATHWAYS, _CHUNK * _PATHWAYS)],
            out_sems[b],
        )
    out_descs[0].wait()
    if _NUM_CHUNKS > 1:
        out_descs[1].wait()


def _tc_body(x_ref, o_ref):
    r = jax.lax.broadcasted_iota(jnp.int32, (_FEATURES, _PATHWAYS), 0)
    c = jax.lax.broadcasted_iota(jnp.int32, (_FEATURES, _PATHWAYS), 1)
    pool = jnp.where(r // _GROUP == c, 1.0 / _GROUP, 0.0).astype(jnp.float32)
    o_ref[...] = jnp.dot(x_ref[...], pool, preferred_element_type=jnp.float32)


@jax.jit
def kernel(gene_set_features):
    mesh = plsc.VectorSubcoreMesh(core_axis_name="c", subcore_axis_name="s")
    run = functools.partial(
        pl.kernel,
        out_type=jax.ShapeDtypeStruct((_SC_ROWS * _PATHWAYS,), jnp.float32),
        mesh=mesh,
        scratch_types=[
            pltpu.VMEM((_CHUNK * _FEATURES,), jnp.float32),
            pltpu.VMEM((_CHUNK * _FEATURES,), jnp.float32),
            pltpu.VMEM((_CHUNK * _PATHWAYS,), jnp.float32),
            pltpu.VMEM((_CHUNK * _PATHWAYS,), jnp.float32),
            pltpu.SemaphoreType.DMA,
            pltpu.SemaphoreType.DMA,
            pltpu.SemaphoreType.DMA,
            pltpu.SemaphoreType.DMA,
        ],
        compiler_params=pltpu.CompilerParams(needs_layout_passes=False),
    )(_sc_body)

    x4 = gene_set_features.reshape(_BATCH // 8, 8, _FEATURES // 128, 128)
    x_flat = x4.transpose(0, 2, 1, 3).reshape(-1)

    sc_flat = run(x_flat)
    sc_part = sc_flat.reshape(_SC_ROWS, _PATHWAYS)

    tc_rows = _BATCH - _SC_ROWS
    tc_part = pl.pallas_call(
        _tc_body,
        grid=(tc_rows // _TC_BLOCK,),
        in_specs=[
            pl.BlockSpec(
                (_TC_BLOCK, _FEATURES),
                lambda i: (i + _SC_ROWS // _TC_BLOCK, 0),
            )
        ],
        out_specs=pl.BlockSpec((_TC_BLOCK, _PATHWAYS), lambda i: (i, 0)),
        out_shape=jax.ShapeDtypeStruct((tc_rows, _PATHWAYS), jnp.float32),
    )(gene_set_features)

    return jnp.concatenate([sc_part, tc_part], axis=0)


# speedup vs baseline: 3.2301x; 1.1326x over previous
import functools

import jax
import jax.numpy as jnp
from jax import lax
from jax.experimental import pallas as pl
from jax.experimental.pallas import tpu as pltpu
from jax.experimental.pallas import tpu_sc as plsc

_BATCH = 16384
_FEATURES = 512
_PATHWAYS = 64
_GROUP = 8

_SC_ROWS = 4096
_NUM_CORES = 2
_NUM_SUBCORES = 16
_NUM_WORKERS = _NUM_CORES * _NUM_SUBCORES
_ROWS_PER_WORKER = _SC_ROWS // _NUM_WORKERS
_CHUNK = 64
_NUM_CHUNKS = _ROWS_PER_WORKER // _CHUNK
_TC_BLOCK = 4096
_LANES = 16
_QVECS = _PATHWAYS // _LANES


def _sc_body(x_hbm, out_hbm, in_v0, in_v1, out_v0, out_v1, si0, si1, so0, so1):
    wid = lax.axis_index("s") * _NUM_CORES + lax.axis_index("c")
    row0 = wid * _ROWS_PER_WORKER

    in_bufs = [in_v0, in_v1]
    out_bufs = [out_v0, out_v1]
    in_sems = [si0, si1]
    out_sems = [so0, so1]

    lane = lax.iota(jnp.int32, _LANES)
    col_idx = [
        [lane * _GROUP + (q * 1024 + k) for k in range(_GROUP)]
        for q in range(_QVECS)
    ]

    def start_in(c):
        rowa = row0 + c * _CHUNK
        return pltpu.async_copy(
            x_hbm.at[pl.ds(rowa * _FEATURES, _CHUNK * _FEATURES)],
            in_bufs[c & 1],
            in_sems[c & 1],
        )

    in_descs = [start_in(0), None]
    out_descs = [None, None]
    for c in range(_NUM_CHUNKS):
        b = c & 1
        if c + 1 < _NUM_CHUNKS:
            in_descs[1 - b] = start_in(c + 1)
        in_descs[b].wait()
        if out_descs[b] is not None:
            out_descs[b].wait()
        in_v = in_bufs[b]
        out_v = out_bufs[b]

        @plsc.parallel_loop(0, _CHUNK, 1, unroll=4)
        def do_row(r):
            rbase = (r // 8) * 4096 + (r % 8) * 128
            in_row = in_v.at[pl.ds(rbase, 3200)]
            for q in range(_QVECS):
                acc = plsc.load_gather(in_row, [col_idx[q][0]])
                for k in range(1, _GROUP):
                    acc = acc + plsc.load_gather(in_row, [col_idx[q][k]])
                out_v[pl.ds(r * _PATHWAYS + q * _LANES, _LANES)] = acc * (
                    1.0 / _GROUP
                )

        rowa = row0 + c * _CHUNK
        out_descs[b] = pltpu.async_copy(
            out_v,
            out_hbm.at[pl.ds(rowa * _PATHWAYS, _CHUNK * _PATHWAYS)],
            out_sems[b],
        )
    out_descs[0].wait()
    if _NUM_CHUNKS > 1:
        out_descs[1].wait()


def _tc_body(x_ref, o_ref):
    r = jax.lax.broadcasted_iota(jnp.int32, (_FEATURES, _PATHWAYS), 0)
    c = jax.lax.broadcasted_iota(jnp.int32, (_FEATURES, _PATHWAYS), 1)
    pool = jnp.where(r // _GROUP == c, 1.0 / _GROUP, 0.0).astype(jnp.float32)
    o_ref[...] = jnp.dot(x_ref[...], pool, preferred_element_type=jnp.float32)


@jax.jit
def kernel(gene_set_features):
    mesh = plsc.VectorSubcoreMesh(core_axis_name="c", subcore_axis_name="s")
    run = functools.partial(
        pl.kernel,
        out_type=jax.ShapeDtypeStruct((_SC_ROWS * _PATHWAYS,), jnp.float32),
        mesh=mesh,
        scratch_types=[
            pltpu.VMEM((_CHUNK * _FEATURES,), jnp.float32),
            pltpu.VMEM((_CHUNK * _FEATURES,), jnp.float32),
            pltpu.VMEM((_CHUNK * _PATHWAYS,), jnp.float32),
            pltpu.VMEM((_CHUNK * _PATHWAYS,), jnp.float32),
            pltpu.SemaphoreType.DMA,
            pltpu.SemaphoreType.DMA,
            pltpu.SemaphoreType.DMA,
            pltpu.SemaphoreType.DMA,
        ],
        compiler_params=pltpu.CompilerParams(needs_layout_passes=False),
    )(_sc_body)

    x4 = gene_set_features.reshape(_BATCH // 8, 8, _FEATURES // 128, 128)
    x_flat = x4.transpose(0, 2, 1, 3).reshape(-1)

    sc_flat = run(x_flat)
    sc_part = sc_flat.reshape(_SC_ROWS, _PATHWAYS)

    tc_full = pl.pallas_call(
        _tc_body,
        grid=((_BATCH - _SC_ROWS) // _TC_BLOCK,),
        in_specs=[
            pl.BlockSpec(
                (_TC_BLOCK, _FEATURES),
                lambda i: (i + _SC_ROWS // _TC_BLOCK, 0),
            )
        ],
        out_specs=pl.BlockSpec(
            (_TC_BLOCK, _PATHWAYS), lambda i: (i + _SC_ROWS // _TC_BLOCK, 0)
        ),
        out_shape=jax.ShapeDtypeStruct((_BATCH, _PATHWAYS), jnp.float32),
    )(gene_set_features)

    return jax.lax.dynamic_update_slice(tc_full, sc_part, (0, 0))


# speedup vs baseline: 4.0034x; 1.2394x over previous
import functools

import jax
import jax.numpy as jnp
from jax import lax
from jax.experimental import pallas as pl
from jax.experimental.pallas import tpu as pltpu
from jax.experimental.pallas import tpu_sc as plsc

_BATCH = 16384
_FEATURES = 512
_PATHWAYS = 64
_GROUP = 8

_SC_ROWS = 4096
_NUM_CORES = 2
_NUM_SUBCORES = 16
_NUM_WORKERS = _NUM_CORES * _NUM_SUBCORES
_ROWS_PER_WORKER = _SC_ROWS // _NUM_WORKERS
_CHUNK = 64
_NUM_CHUNKS = _ROWS_PER_WORKER // _CHUNK
_TC_BLOCK = 4096
_LANES = 16
_QVECS = _PATHWAYS // _LANES


def _sc_body(x_hbm, out_hbm, in_v0, in_v1, out_v0, out_v1, si0, si1, so0, so1):
    wid = lax.axis_index("s") * _NUM_CORES + lax.axis_index("c")
    row0 = wid * _ROWS_PER_WORKER

    in_bufs = [in_v0, in_v1]
    out_bufs = [out_v0, out_v1]
    in_sems = [si0, si1]
    out_sems = [so0, so1]

    lane = lax.iota(jnp.int32, _LANES)
    col_idx = [
        [lane * _GROUP + (q * 1024 + k) for k in range(_GROUP)]
        for q in range(_QVECS)
    ]

    def start_in(c):
        rowa = row0 + c * _CHUNK
        return pltpu.async_copy(
            x_hbm.at[pl.ds(rowa * _FEATURES, _CHUNK * _FEATURES)],
            in_bufs[c & 1],
            in_sems[c & 1],
        )

    in_descs = [start_in(0), None]
    out_descs = [None, None]
    for c in range(_NUM_CHUNKS):
        b = c & 1
        if c + 1 < _NUM_CHUNKS:
            in_descs[1 - b] = start_in(c + 1)
        in_descs[b].wait()
        if out_descs[b] is not None:
            out_descs[b].wait()
        in_v = in_bufs[b]
        out_v = out_bufs[b]

        @plsc.parallel_loop(0, _CHUNK, 1, unroll=2)
        def do_row(r):
            rbase = (r // 8) * 4096 + (r % 8) * 128
            in_row = in_v.at[pl.ds(rbase, 3200)]
            for q in range(_QVECS):
                acc = plsc.load_gather(in_row, [col_idx[q][0]])
                for k in range(1, _GROUP):
                    acc = acc + plsc.load_gather(in_row, [col_idx[q][k]])
                out_v[pl.ds(r * _PATHWAYS + q * _LANES, _LANES)] = acc * (
                    1.0 / _GROUP
                )

        rowa = row0 + c * _CHUNK
        out_descs[b] = pltpu.async_copy(
            out_v,
            out_hbm.at[pl.ds(rowa * _PATHWAYS, _CHUNK * _PATHWAYS)],
            out_sems[b],
        )
    out_descs[0].wait()
    if _NUM_CHUNKS > 1:
        out_descs[1].wait()


def _tc_body(x_ref, o_ref):
    r = jax.lax.broadcasted_iota(jnp.int32, (_FEATURES, _PATHWAYS), 0)
    c = jax.lax.broadcasted_iota(jnp.int32, (_FEATURES, _PATHWAYS), 1)
    pool = jnp.where(r // _GROUP == c, 1.0 / _GROUP, 0.0).astype(jnp.float32)
    o_ref[...] = jax.lax.dot_general(
        pool,
        x_ref[...],
        dimension_numbers=(((0,), (1,)), ((), ())),
        preferred_element_type=jnp.float32,
    )


def _kernel_impl(gene_set_features):
    mesh = plsc.VectorSubcoreMesh(core_axis_name="c", subcore_axis_name="s")
    run = functools.partial(
        pl.kernel,
        out_type=jax.ShapeDtypeStruct((_SC_ROWS * _PATHWAYS,), jnp.float32),
        mesh=mesh,
        scratch_types=[
            pltpu.VMEM((_CHUNK * _FEATURES,), jnp.float32),
            pltpu.VMEM((_CHUNK * _FEATURES,), jnp.float32),
            pltpu.VMEM((_CHUNK * _PATHWAYS,), jnp.float32),
            pltpu.VMEM((_CHUNK * _PATHWAYS,), jnp.float32),
            pltpu.SemaphoreType.DMA,
            pltpu.SemaphoreType.DMA,
            pltpu.SemaphoreType.DMA,
            pltpu.SemaphoreType.DMA,
        ],
        compiler_params=pltpu.CompilerParams(needs_layout_passes=False),
    )(_sc_body)

    x4 = gene_set_features.reshape(_BATCH // 8, 8, _FEATURES // 128, 128)
    x_flat = x4.transpose(0, 2, 1, 3).reshape(-1)

    sc_flat = run(x_flat)
    sc_part = sc_flat.reshape(_SC_ROWS, _PATHWAYS)

    tc_full_t = pl.pallas_call(
        _tc_body,
        grid=((_BATCH - _SC_ROWS) // _TC_BLOCK,),
        in_specs=[
            pl.BlockSpec(
                (_TC_BLOCK, _FEATURES),
                lambda i: (i + _SC_ROWS // _TC_BLOCK, 0),
            )
        ],
        out_specs=pl.BlockSpec(
            (_PATHWAYS, _TC_BLOCK), lambda i: (0, i + _SC_ROWS // _TC_BLOCK)
        ),
        out_shape=jax.ShapeDtypeStruct((_PATHWAYS, _BATCH), jnp.float32),
    )(gene_set_features)

    merged_t = jax.lax.dynamic_update_slice(tc_full_t, sc_part.T, (0, 0))
    return merged_t.T


kernel = jax.jit(_kernel_impl)
